# baseline (speedup 1.0000x reference)
import jax
import jax.numpy as jnp
from jax import lax
from jax.experimental import pallas as pl
from jax.experimental.pallas import tpu as pltpu
from jax.experimental.pallas import tpu_sc as plsc

B = 1024
NNEG = 128
HID = 64
G = 2
DIM = 65
GAMMA = 12.0
WPAD = 72
RPAD = 80

NC, NS, L = 2, 16, 16
NW = NC * NS
BPW = B // NW
NR = G * BPW
F32 = jnp.float32
I32 = jnp.int32


def _splat_i32(x):
    return jnp.full((L,), x, I32)


def _splat_f32(x):
    return jnp.full((L,), x, F32)


def _body(ent_h, rel_h, w1_h, b1_h, sw_h, aidx_h, ridx_h, pidx_h,
          nidxa_h, nidxb_h, bidx_h,
          pos_out_h, neg_out_h, w_out_h,
          W1_v, b1_v, sw_v, bidx_v,
          IDXA, IDXR, IDXP, SA, SR, PV, XV, Q_v, HW_v, WQ_v,
          IDX0A, IDX0B, IDX1A, IDX1B, G0, G1, OUT_v, PO_v, WO_v,
          sem_a, sem_r, sem_p, sem0, sem1):
    wid = lax.axis_index("s") * NC + lax.axis_index("c")
    base = wid * BPW
    iota = lax.iota(I32, L)

    pltpu.sync_copy(aidx_h.at[pl.ds(2 * base, NR)], IDXA)
    pltpu.sync_copy(ridx_h.at[pl.ds(2 * base, NR)], IDXR)
    pltpu.sync_copy(pidx_h.at[pl.ds(2 * base, NR)], IDXP)

    pltpu.sync_copy(ent_h.at[IDXA], SA)
    pltpu.sync_copy(rel_h.at[IDXR], SR)
    pltpu.sync_copy(ent_h.at[IDXP], PV)

    pltpu.sync_copy(w1_h, W1_v)
    pltpu.sync_copy(b1_h, b1_v)
    pltpu.sync_copy(sw_h, sw_v)
    pltpu.sync_copy(bidx_h.at[pl.ds(base, BPW)], bidx_v)

    for j in range(BPW // L):
        bv = bidx_v[pl.ds(j * L, L)]
        WO_v[pl.ds(j * L, L)] = plsc.load_gather(sw_v, [bv])
    pltpu.sync_copy(WO_v, w_out_h.at[pl.ds(base, BPW)])

    def xrow(r, _):
        for j in range(RPAD // L):
            XV[r, pl.ds(j * L, L)] = (SA[r, pl.ds(j * L, L)]
                                      + SR[r, pl.ds(j * L, L)])
        return 0

    lax.fori_loop(0, NR, xrow, 0)

    b1r = [b1_v[pl.ds(j * L, L)] for j in range(HID // L)]
    b1w = plsc.load_gather(b1_v, [_splat_i32(DIM - 1)])

    def mlp4(r4, _):
        r0 = r4 * 4
        xs = [[XV[r0 + k, pl.ds(j * L, L)] for j in range(4)] for k in range(4)]
        xt = [plsc.load_gather(XV, [_splat_i32(r0 + k), _splat_i32(HID)])
              for k in range(4)]
        acc = [[jnp.zeros((L,), F32) for _ in range(4)] for _ in range(4)]

        def dblk(d16, acc):
            xsel = [jnp.where(d16 == 0, xs[k][0],
                    jnp.where(d16 == 1, xs[k][1],
                    jnp.where(d16 == 2, xs[k][2], xs[k][3])))
                    for k in range(4)]
            dbase = d16 * L
            for kk in range(L):
                w = [W1_v[dbase + kk, pl.ds(j * L, L)] for j in range(4)]
                for k in range(4):
                    xb = _splat_f32(xsel[k][kk])
                    for j in range(4):
                        acc[k][j] = acc[k][j] + xb * w[j]
            return acc

        acc = lax.fori_loop(0, 4, dblk, acc)
        w64 = [W1_v[HID, pl.ds(j * L, L)] for j in range(4)]
        for k in range(4):
            for j in range(4):
                q = acc[k][j] + xt[k] * w64[j] + b1r[j]
                Q_v[r0 + k, pl.ds(j * L, L)] = jnp.clip(q, 0.05, 1e9)
        return 0

    lax.fori_loop(0, NR // 4, mlp4, 0)

    for j in range(NR // L):
        rv = iota + j * L

        def awbody(d, aw):
            xb = plsc.load_gather(XV, [rv, _splat_i32(d)])
            wv = plsc.load_gather(W1_v, [_splat_i32(d), _splat_i32(HID)])
            return aw + xb * wv

        aw = lax.fori_loop(0, DIM, awbody, jnp.zeros((L,), F32))
        HW_v[pl.ds(j * L, L)] = jnp.clip(aw + b1w, 0.05, 1e9)

    for j in range(BPW // L):
        iv = iota + j * L
        h0 = plsc.load_gather(HW_v, [iv * 2])
        h1 = plsc.load_gather(HW_v, [iv * 2 + 1])
        m = jnp.maximum(h0, h1)
        e0 = jnp.exp(h0 - m)
        e1 = jnp.exp(h1 - m)
        s = e0 + e1
        WQ_v[pl.ds(j * L, L)] = e0 / s
        WQ_v[pl.ds(BPW + j * L, L)] = e1 / s

    for j in range(BPW // L):
        iv = iota + j * L
        accs = (jnp.zeros((L,), F32), jnp.zeros((L,), F32))

        def phbody(h, accs):
            a0, a1 = accs
            hv = _splat_i32(h)
            p0 = plsc.load_gather(PV, [iv * 2, hv])
            q0 = plsc.load_gather(Q_v, [iv * 2, hv])
            p1 = plsc.load_gather(PV, [iv * 2 + 1, hv])
            q1 = plsc.load_gather(Q_v, [iv * 2 + 1, hv])
            return a0 + jnp.abs(p0 - q0), a1 + jnp.abs(p1 - q1)

        d0, d1 = lax.fori_loop(0, HID, phbody, accs)
        w0 = WQ_v[pl.ds(j * L, L)]
        w1 = WQ_v[pl.ds(BPW + j * L, L)]
        logit = GAMMA - (w0 * d0 + w1 * d1)
        plsc.store_scatter(PO_v, [iv, _splat_i32(0)], logit)
    pltpu.sync_copy(PO_v, pos_out_h.at[pl.ds(base, BPW)])

    def compute(i, g_ref):
        q0 = [Q_v[2 * i, pl.ds(j * L, L)] for j in range(4)]
        q1 = [Q_v[2 * i + 1, pl.ds(j * L, L)] for j in range(4)]
        w0 = plsc.load_gather(WQ_v, [_splat_i32(i)])
        w1 = plsc.load_gather(WQ_v, [_splat_i32(BPW + i)])
        acc = [jnp.zeros((L,), F32) for _ in range(2 * (NNEG // L))]

        def hblk(h16, acc):
            acc = list(acc)
            q0s = jnp.where(h16 == 0, q0[0],
                  jnp.where(h16 == 1, q0[1],
                  jnp.where(h16 == 2, q0[2], q0[3])))
            q1s = jnp.where(h16 == 0, q1[0],
                  jnp.where(h16 == 1, q1[1],
                  jnp.where(h16 == 2, q1[2], q1[3])))
            hh = h16 * L
            for kk in range(L):
                hv = _splat_i32(hh + kk)
                q0b = _splat_f32(q0s[kk])
                q1b = _splat_f32(q1s[kk])
                for ng in range(NNEG // L):
                    r0v = iota + ng * L
                    r1v = iota + (ng * L + NNEG)
                    c0 = plsc.load_gather(g_ref, [r0v, hv])
                    c1 = plsc.load_gather(g_ref, [r1v, hv])
                    acc[2 * ng] = acc[2 * ng] + jnp.abs(c0 - q0b)
                    acc[2 * ng + 1] = acc[2 * ng + 1] + jnp.abs(c1 - q1b)
            return tuple(acc)

        acc = lax.fori_loop(0, HID // L, hblk, tuple(acc))
        for ng in range(NNEG // L):
            logit = GAMMA - (w0 * acc[2 * ng] + w1 * acc[2 * ng + 1])
            OUT_v[i, pl.ds(ng * L, L)] = logit

    def pipe(i, _):
        pltpu.sync_copy(nidxa_h.at[base + i], IDX0A)
        pltpu.sync_copy(nidxb_h.at[base + i], IDX0B)
        pltpu.sync_copy(ent_h.at[IDX0A], G0.at[pl.ds(0, NNEG)])
        pltpu.sync_copy(ent_h.at[IDX0B], G0.at[pl.ds(NNEG, NNEG)])
        compute(i, G0)
        return 0

    lax.fori_loop(0, BPW, pipe, 0)
    pltpu.sync_copy(OUT_v, neg_out_h.at[pl.ds(base, BPW)])


def kernel(input_entity_embedding, rel_table, W1, b1, subsampling_weight,
           positive_sample, negative_sample, queries, batch_idxs):
    ent1 = jnp.pad(input_entity_embedding.reshape(-1, DIM),
                   ((0, 0), (0, RPAD - DIM)))
    rel1 = jnp.pad(rel_table.reshape(-1, DIM),
                   ((0, 0), (0, RPAD - DIM)))
    w1p = jnp.pad(W1, ((0, 0), (0, WPAD - DIM)))
    two_g = jnp.arange(G, dtype=I32)[None, :]
    aidx = (queries[:, :1].astype(I32) * 4 + two_g).reshape(-1)
    ridx = (queries[:, 1:2].astype(I32) * 4 + two_g).reshape(-1)
    pidx = (positive_sample[:, None].astype(I32) * 4 + two_g).reshape(-1)
    nidxa = negative_sample.astype(I32) * 4
    nidxb = nidxa + 1
    bidx = batch_idxs.astype(I32)

    mesh = plsc.VectorSubcoreMesh(core_axis_name="c", subcore_axis_name="s")
    f = pl.kernel(
        _body,
        out_type=(
            jax.ShapeDtypeStruct((B, 1), F32),
            jax.ShapeDtypeStruct((B, NNEG), F32),
            jax.ShapeDtypeStruct((B,), F32),
        ),
        mesh=mesh,
        compiler_params=pltpu.CompilerParams(
            needs_layout_passes=False, use_tc_tiling_on_sc=False),
        scratch_types=[
            pltpu.VMEM((DIM, WPAD), F32),
            pltpu.VMEM((DIM,), F32),
            pltpu.VMEM((B,), F32),
            pltpu.VMEM((BPW,), I32),
            pltpu.VMEM((NR,), I32),
            pltpu.VMEM((NR,), I32),
            pltpu.VMEM((NR,), I32),
            pltpu.VMEM((NR, RPAD), F32),
            pltpu.VMEM((NR, RPAD), F32),
            pltpu.VMEM((NR, RPAD), F32),
            pltpu.VMEM((NR, RPAD), F32),
            pltpu.VMEM((NR, HID), F32),
            pltpu.VMEM((NR,), F32),
            pltpu.VMEM((NR,), F32),
            pltpu.VMEM((NNEG,), I32),
            pltpu.VMEM((NNEG,), I32),
            pltpu.VMEM((NNEG,), I32),
            pltpu.VMEM((NNEG,), I32),
            pltpu.VMEM((2 * NNEG, RPAD), F32),
            pltpu.VMEM((2 * NNEG, RPAD), F32),
            pltpu.VMEM((BPW, NNEG), F32),
            pltpu.VMEM((BPW, 1), F32),
            pltpu.VMEM((BPW,), F32),
            pltpu.SemaphoreType.DMA,
            pltpu.SemaphoreType.DMA,
            pltpu.SemaphoreType.DMA,
            pltpu.SemaphoreType.DMA,
            pltpu.SemaphoreType.DMA,
        ],
    )
    return f(ent1, rel1, w1p, b1, subsampling_weight,
             aidx, ridx, pidx, nidxa, nidxb, bidx)


# speedup vs baseline: 1.0097x; 1.0097x over previous
import jax
import jax.numpy as jnp
from jax import lax
from jax.experimental import pallas as pl
from jax.experimental.pallas import tpu as pltpu
from jax.experimental.pallas import tpu_sc as plsc

B = 1024
NNEG = 128
HID = 64
G = 2
DIM = 65
GAMMA = 12.0
WPAD = 72
RPAD = 80
GR = 72

NC, NS, L = 2, 16, 16
NW = NC * NS
BPW = B // NW
NR = G * BPW
F32 = jnp.float32
I32 = jnp.int32


def _splat_i32(x):
    return jnp.full((L,), x, I32)


def _splat_f32(x):
    return jnp.full((L,), x, F32)


_PADBLK = 4000


def _pad_body_tc(x_ref, o_ref):
    o_ref[:, :HID] = x_ref[:, :HID]
    tail = jnp.concatenate(
        [x_ref[:, HID:DIM], jnp.zeros((_PADBLK, RPAD - DIM), F32)], axis=1)
    o_ref[:, HID:RPAD] = tail


def _pad_rows_tc(x):
    n = x.shape[0]
    return pl.pallas_call(
        _pad_body_tc,
        grid=(n // _PADBLK,),
        in_specs=[pl.BlockSpec((_PADBLK, DIM), lambda i: (i, 0))],
        out_specs=pl.BlockSpec((_PADBLK, RPAD), lambda i: (i, 0)),
        out_shape=jax.ShapeDtypeStruct((n, RPAD), F32),
    )(x)


def _body(ent_h, rel_h, w1_h, b1_h, sw_h, aidx_h, ridx_h, pidx_h,
          nidxa_h, nidxb_h, bidx_h,
          pos_out_h, neg_out_h, w_out_h,
          W1_v, b1_v, sw_v, bidx_v,
          IDXA, IDXR, IDXP, SA, SR, PV, XV, Q_v, HW_v, WQ_v,
          NIA, NIB, G0, G1, OUT_v, PO_v, WO_v,
          sem_a, sem_r, sem_p, sem0, sem1, sem2, sem3):
    wid = lax.axis_index("s") * NC + lax.axis_index("c")
    base = wid * BPW
    iota = lax.iota(I32, L)

    pltpu.sync_copy(aidx_h.at[pl.ds(2 * base, NR)], IDXA)
    pltpu.sync_copy(ridx_h.at[pl.ds(2 * base, NR)], IDXR)
    pltpu.sync_copy(pidx_h.at[pl.ds(2 * base, NR)], IDXP)

    cpa = pltpu.async_copy(ent_h.at[IDXA], SA, sem_a)
    cpr = pltpu.async_copy(rel_h.at[IDXR], SR, sem_r)
    cpp = pltpu.async_copy(ent_h.at[IDXP], PV, sem_p)

    pltpu.sync_copy(w1_h, W1_v)
    pltpu.sync_copy(b1_h, b1_v)
    pltpu.sync_copy(sw_h, sw_v)
    pltpu.sync_copy(bidx_h.at[pl.ds(base, BPW)], bidx_v)
    pltpu.sync_copy(nidxa_h.at[pl.ds(base, BPW)], NIA)
    pltpu.sync_copy(nidxb_h.at[pl.ds(base, BPW)], NIB)

    for j in range(BPW // L):
        bv = bidx_v[pl.ds(j * L, L)]
        WO_v[pl.ds(j * L, L)] = plsc.load_gather(sw_v, [bv])
    pltpu.sync_copy(WO_v, w_out_h.at[pl.ds(base, BPW)])

    cpa.wait()
    cpr.wait()
    for j in range(NR // L):
        rv = iota + j * L
        for c in range(DIM):
            xa = plsc.load_gather(SA, [rv, _splat_i32(c)])
            xr = plsc.load_gather(SR, [rv, _splat_i32(c)])
            plsc.store_scatter(XV, [rv, _splat_i32(c)], xa + xr)

    b1r = [b1_v[pl.ds(j * L, L)] for j in range(HID // L)]
    b1w = plsc.load_gather(b1_v, [_splat_i32(DIM - 1)])

    def mlp4(r4, _):
        r0 = r4 * 4
        xs = [[XV[r0 + k, pl.ds(j * L, L)] for j in range(4)] for k in range(4)]
        xt = [plsc.load_gather(XV, [_splat_i32(r0 + k), _splat_i32(HID)])
              for k in range(4)]
        acc = [[jnp.zeros((L,), F32) for _ in range(4)] for _ in range(4)]

        def dblk(d16, acc):
            xsel = [jnp.where(d16 == 0, xs[k][0],
                    jnp.where(d16 == 1, xs[k][1],
                    jnp.where(d16 == 2, xs[k][2], xs[k][3])))
                    for k in range(4)]
            dbase = d16 * L
            for kk in range(L):
                w = [W1_v[dbase + kk, pl.ds(j * L, L)] for j in range(4)]
                for k in range(4):
                    xb = _splat_f32(xsel[k][kk])
                    for j in range(4):
                        acc[k][j] = acc[k][j] + xb * w[j]
            return acc

        acc = lax.fori_loop(0, 4, dblk, acc)
        w64 = [W1_v[HID, pl.ds(j * L, L)] for j in range(4)]
        for k in range(4):
            for j in range(4):
                q = acc[k][j] + xt[k] * w64[j] + b1r[j]
                Q_v[r0 + k, pl.ds(j * L, L)] = jnp.clip(q, 0.05, 1e9)
        return 0

    lax.fori_loop(0, NR // 4, mlp4, 0)

    for j in range(NR // L):
        rv = iota + j * L

        def awbody(d, aw):
            xb = plsc.load_gather(XV, [rv, _splat_i32(d)])
            wv = plsc.load_gather(W1_v, [_splat_i32(d), _splat_i32(HID)])
            return aw + xb * wv

        aw = lax.fori_loop(0, DIM, awbody, jnp.zeros((L,), F32))
        HW_v[pl.ds(j * L, L)] = jnp.clip(aw + b1w, 0.05, 1e9)

    for j in range(BPW // L):
        iv = iota + j * L
        h0 = plsc.load_gather(HW_v, [iv * 2])
        h1 = plsc.load_gather(HW_v, [iv * 2 + 1])
        m = jnp.maximum(h0, h1)
        e0 = jnp.exp(h0 - m)
        e1 = jnp.exp(h1 - m)
        s = e0 + e1
        WQ_v[pl.ds(j * L, L)] = e0 / s
        WQ_v[pl.ds(BPW + j * L, L)] = e1 / s

    cpp.wait()
    for j in range(BPW // L):
        iv = iota + j * L
        accs = (jnp.zeros((L,), F32), jnp.zeros((L,), F32))

        def phbody(h, accs):
            a0, a1 = accs
            hv = _splat_i32(h)
            p0 = plsc.load_gather(PV, [iv * 2, hv])
            q0 = plsc.load_gather(Q_v, [iv * 2, hv])
            p1 = plsc.load_gather(PV, [iv * 2 + 1, hv])
            q1 = plsc.load_gather(Q_v, [iv * 2 + 1, hv])
            return a0 + jnp.abs(p0 - q0), a1 + jnp.abs(p1 - q1)

        d0, d1 = lax.fori_loop(0, HID, phbody, accs)
        w0 = WQ_v[pl.ds(j * L, L)]
        w1 = WQ_v[pl.ds(BPW + j * L, L)]
        logit = GAMMA - (w0 * d0 + w1 * d1)
        plsc.store_scatter(PO_v, [iv, _splat_i32(0)], logit)
    pltpu.sync_copy(PO_v, pos_out_h.at[pl.ds(base, BPW)])

    def compute(i, g_ref):
        q0 = [Q_v[2 * i, pl.ds(j * L, L)] for j in range(4)]
        q1 = [Q_v[2 * i + 1, pl.ds(j * L, L)] for j in range(4)]
        w0 = plsc.load_gather(WQ_v, [_splat_i32(i)])
        w1 = plsc.load_gather(WQ_v, [_splat_i32(BPW + i)])
        acc = [jnp.zeros((L,), F32) for _ in range(2 * (NNEG // L))]

        def hblk(h16, acc):
            acc = list(acc)
            q0s = jnp.where(h16 == 0, q0[0],
                  jnp.where(h16 == 1, q0[1],
                  jnp.where(h16 == 2, q0[2], q0[3])))
            q1s = jnp.where(h16 == 0, q1[0],
                  jnp.where(h16 == 1, q1[1],
                  jnp.where(h16 == 2, q1[2], q1[3])))
            hh = h16 * L
            for kk in range(L):
                hv = _splat_i32(hh + kk)
                q0b = _splat_f32(q0s[kk])
                q1b = _splat_f32(q1s[kk])
                for ng in range(NNEG // L):
                    r0v = iota + ng * L
                    r1v = iota + (ng * L + NNEG)
                    c0 = plsc.load_gather(g_ref, [r0v, hv])
                    c1 = plsc.load_gather(g_ref, [r1v, hv])
                    acc[2 * ng] = acc[2 * ng] + jnp.abs(c0 - q0b)
                    acc[2 * ng + 1] = acc[2 * ng + 1] + jnp.abs(c1 - q1b)
            return tuple(acc)

        acc = lax.fori_loop(0, HID // L, hblk, tuple(acc))
        for ng in range(NNEG // L):
            logit = GAMMA - (w0 * acc[2 * ng] + w1 * acc[2 * ng + 1])
            OUT_v[i, pl.ds(ng * L, L)] = logit

    def fire(i, g_ref, s0, s1):
        pltpu.async_copy(ent_h.at[NIA.at[i]], g_ref.at[pl.ds(0, NNEG)], s0)
        pltpu.async_copy(ent_h.at[NIB.at[i]], g_ref.at[pl.ds(NNEG, NNEG)], s1)

    def drain(i, g_ref, s0, s1):
        pltpu.make_async_copy(
            ent_h.at[NIA.at[i]], g_ref.at[pl.ds(0, NNEG)], s0).wait()
        pltpu.make_async_copy(
            ent_h.at[NIB.at[i]], g_ref.at[pl.ds(NNEG, NNEG)], s1).wait()

    fire(0, G0, sem0, sem1)

    def pipe(k, _):
        i = k * 2
        fire(i + 1, G1, sem2, sem3)
        drain(i, G0, sem0, sem1)
        compute(i, G0)

        @pl.when(k < BPW // 2 - 1)
        def _fire_next():
            fire(i + 2, G0, sem0, sem1)

        drain(i + 1, G1, sem2, sem3)
        compute(i + 1, G1)
        return 0

    lax.fori_loop(0, BPW // 2, pipe, 0)
    pltpu.sync_copy(OUT_v, neg_out_h.at[pl.ds(base, BPW)])


def kernel(input_entity_embedding, rel_table, W1, b1, subsampling_weight,
           positive_sample, negative_sample, queries, batch_idxs):
    ent1 = _pad_rows_tc(input_entity_embedding.reshape(-1, DIM))
    rel1 = jnp.pad(rel_table.reshape(-1, DIM),
                   ((0, 0), (0, RPAD - DIM)))
    w1p = jnp.pad(W1, ((0, WPAD - DIM), (0, WPAD - DIM)))
    b1p = jnp.pad(b1, (0, RPAD - DIM))
    two_g = jnp.arange(G, dtype=I32)[None, :]
    aidx = (queries[:, :1].astype(I32) * 4 + two_g).reshape(-1)
    ridx = (queries[:, 1:2].astype(I32) * 4 + two_g).reshape(-1)
    pidx = (positive_sample[:, None].astype(I32) * 4 + two_g).reshape(-1)
    nidxa = negative_sample.astype(I32) * 4
    nidxb = nidxa + 1
    bidx = batch_idxs.astype(I32)

    mesh = plsc.VectorSubcoreMesh(core_axis_name="c", subcore_axis_name="s")
    f = pl.kernel(
        _body,
        out_type=(
            jax.ShapeDtypeStruct((B, 1), F32),
            jax.ShapeDtypeStruct((B, NNEG), F32),
            jax.ShapeDtypeStruct((B,), F32),
        ),
        mesh=mesh,
        compiler_params=pltpu.CompilerParams(
            needs_layout_passes=False, use_tc_tiling_on_sc=False),
        scratch_types=[
            pltpu.VMEM((WPAD, WPAD), F32),
            pltpu.VMEM((RPAD,), F32),
            pltpu.VMEM((B,), F32),
            pltpu.VMEM((BPW,), I32),
            pltpu.VMEM((NR,), I32),
            pltpu.VMEM((NR,), I32),
            pltpu.VMEM((NR,), I32),
            pltpu.VMEM((NR, RPAD), F32),
            pltpu.VMEM((NR, RPAD), F32),
            pltpu.VMEM((NR, RPAD), F32),
            pltpu.VMEM((NR, WPAD), F32),
            pltpu.VMEM((NR, HID), F32),
            pltpu.VMEM((NR,), F32),
            pltpu.VMEM((NR,), F32),
            pltpu.VMEM((BPW, NNEG), I32),
            pltpu.VMEM((BPW, NNEG), I32),
            pltpu.VMEM((2 * NNEG, RPAD), F32),
            pltpu.VMEM((2 * NNEG, RPAD), F32),
            pltpu.VMEM((BPW, NNEG), F32),
            pltpu.VMEM((BPW, 1), F32),
            pltpu.VMEM((BPW,), F32),
            pltpu.SemaphoreType.DMA,
            pltpu.SemaphoreType.DMA,
            pltpu.SemaphoreType.DMA,
            pltpu.SemaphoreType.DMA,
            pltpu.SemaphoreType.DMA,
            pltpu.SemaphoreType.DMA,
            pltpu.SemaphoreType.DMA,
        ],
    )
    return f(ent1, rel1, w1p, b1p, subsampling_weight,
             aidx, ridx, pidx, nidxa, nidxb, bidx)
